# baseline (speedup 1.0000x reference)
import functools

import jax
import jax.numpy as jnp
from jax import lax
from jax.experimental import pallas as pl
from jax.experimental.pallas import tpu as pltpu
from jax.experimental.pallas import tpu_sc as plsc

_NUM_CORES = 2
_NUM_SUBCORES = 16
_NW = _NUM_CORES * _NUM_SUBCORES
_L = 16


def _repack_split(n_tok, rows_blk=1024):
    n_blk = -(-(n_tok // 2) // rows_blk)
    return n_blk * rows_blk, n_blk


def _repack_table(tok_t, *, n_tok, d, rows_blk=1024):
    hp, n_blk = _repack_split(n_tok, rows_blk)
    max_blk = n_tok // rows_blk

    def body(lo_ref, hi_ref, out_ref):
        out_ref[:, 0:d] = jnp.transpose(lo_ref[...], (1, 0))
        out_ref[:, d:2 * d] = jnp.transpose(hi_ref[...], (1, 0))

    return pl.pallas_call(
        body,
        grid=(n_blk,),
        in_specs=[
            pl.BlockSpec((d, rows_blk), lambda i: (0, i)),
            pl.BlockSpec(
                (d, rows_blk),
                lambda i, n=n_blk, m=max_blk: (0, jnp.minimum(i + n, m))),
        ],
        out_specs=pl.BlockSpec((rows_blk, 2 * d), lambda i: (i, 0)),
        out_shape=jax.ShapeDtypeStruct((hp, 2 * d), jnp.float32),
    )(tok_t, tok_t)


@functools.partial(jax.jit, static_argnames=("batch", "seq_len", "d", "hp"))
def _emb_lookup(idx_flat, tok128, pos_table, *, batch, seq_len, d, hp):
    seqs_per_w = batch // _NW
    n_pairs = seqs_per_w // 2
    d_vregs = d // _L
    idx_vregs = -(-seq_len // _L)
    n_gather = idx_vregs * _L

    mesh = plsc.VectorSubcoreMesh(core_axis_name="c", subcore_axis_name="s")

    @functools.partial(
        pl.kernel,
        mesh=mesh,
        out_type=jax.ShapeDtypeStruct((batch, seq_len, d), jnp.float32),
        scratch_types=[
            pltpu.VMEM((n_gather + _L,), jnp.int32),
            pltpu.VMEM((n_gather + _L,), jnp.int32),
            pltpu.VMEM((n_gather,), jnp.int32),
            pltpu.VMEM((n_gather,), jnp.int32),
            pltpu.VMEM((n_gather, 2 * d), jnp.float32),
            pltpu.VMEM((n_gather, 2 * d), jnp.float32),
            pltpu.VMEM((seq_len, d), jnp.float32),
            pltpu.VMEM((seq_len, d), jnp.float32),
            pltpu.VMEM((seq_len, d), jnp.float32),
            pltpu.SemaphoreType.DMA,
            pltpu.SemaphoreType.DMA,
            pltpu.SemaphoreType.DMA,
            pltpu.SemaphoreType.DMA,
        ],
    )
    def body(idx_hbm, tok_hbm, pos_hbm, out_hbm,
             raw_a, raw_b, gl_a, gl_b, gb_a, gb_b, sb_a, sb_b, pos_v,
             gsem_a, gsem_b, ssem_a, ssem_b):
        cid = lax.axis_index("c")
        sid = lax.axis_index("s")
        wid = sid * _NUM_CORES + cid
        base_seq = wid * seqs_per_w

        pltpu.sync_copy(pos_hbm, pos_v)

        def start_gather(q, raw_r, gl_r, gb_r, gsem):
            tok0 = (base_seq + q) * seq_len
            pltpu.sync_copy(idx_hbm.at[pl.ds(tok0, seq_len)],
                            raw_r.at[pl.ds(0, seq_len)])
            for k in range(idx_vregs):
                sl = pl.ds(k * _L, _L)
                raw = raw_r[sl]
                g = jnp.where(raw >= hp, raw - hp, raw)
                gl_r[sl] = jnp.minimum(jnp.maximum(g, 0), hp - 1)
            pltpu.async_copy(tok_hbm.at[gl_r], gb_r, gsem)

        def wait_gather(gl_r, gb_r, gsem):
            pltpu.make_async_copy(tok_hbm.at[gl_r], gb_r, gsem).wait()

        def select_add(raw_r, gb_r, sb_r):
            def jj_body(jj, carry):
                pair_raw = raw_r[pl.ds(2 * jj, _L)]
                for half in range(2):
                    s = 2 * jj + half
                    p = (pair_raw[half] >= hp).astype(jnp.int32) * d
                    for c in range(d_vregs):
                        sl = pl.ds(c * _L, _L)
                        sb_r[s, sl] = (gb_r[s, pl.ds(p + c * _L, _L)]
                                       + pos_v[s, sl])
                return carry
            lax.fori_loop(0, seq_len // 2, jj_body, 0, unroll=4)

        def start_store(q, sb_r, ssem):
            pltpu.async_copy(sb_r, out_hbm.at[base_seq + q], ssem)

        def wait_store(q, sb_r, ssem):
            pltpu.make_async_copy(
                sb_r, out_hbm.at[base_seq + q], ssem).wait()

        start_gather(0, raw_a, gl_a, gb_a, gsem_a)

        def pair_body(h, carry):
            qa = 2 * h
            qb = 2 * h + 1

            start_gather(qb, raw_b, gl_b, gb_b, gsem_b)

            wait_gather(gl_a, gb_a, gsem_a)

            @pl.when(h > 0)
            def _():
                wait_store(qa - 2, sb_a, ssem_a)
            select_add(raw_a, gb_a, sb_a)
            start_store(qa, sb_a, ssem_a)

            @pl.when(h + 1 < n_pairs)
            def _():
                start_gather(qa + 2, raw_a, gl_a, gb_a, gsem_a)

            wait_gather(gl_b, gb_b, gsem_b)

            @pl.when(h > 0)
            def _():
                wait_store(qb - 2, sb_b, ssem_b)
            select_add(raw_b, gb_b, sb_b)
            start_store(qb, sb_b, ssem_b)
            return carry

        lax.fori_loop(0, n_pairs, pair_body, 0)

        wait_store(seqs_per_w - 2, sb_a, ssem_a)
        wait_store(seqs_per_w - 1, sb_b, ssem_b)

    return body(idx_flat, tok128, pos_table)


def kernel(inputs, token_table, pos_table):
    b, s = inputs.shape
    n_tok, d = token_table.shape
    idx_flat = inputs.reshape(-1).astype(jnp.int32)
    hp, _ = _repack_split(n_tok)
    tok128 = _repack_table(token_table.T, n_tok=n_tok, d=d)
    return _emb_lookup(idx_flat, tok128, pos_table,
                       batch=b, seq_len=s, d=d, hp=hp)
